# Initial kernel scaffold; baseline (speedup 1.0000x reference)
import jax
import jax.numpy as jnp
from jax.experimental import pallas as pl


def kernel(x):
    raise NotImplementedError("write your pallas kernel here")


# speedup vs baseline: 2670.7745x; 2670.7745x over previous
import functools

import jax
import jax.numpy as jnp
from jax import lax
from jax.experimental import pallas as pl
from jax.experimental.pallas import tpu as pltpu
from jax.experimental.pallas import tpu_sc as plsc

H = W = 512
NPIX = H * W
NSAMP = 32
CHUNK = 16384
NCHUNK = NPIX // CHUNK
LANES = 16
UNROLL = 8
INNER = CHUNK // (LANES * UNROLL)

_mesh = plsc.VectorSubcoreMesh(core_axis_name="c", subcore_axis_name="s")


@functools.partial(
    pl.kernel,
    mesh=_mesh,
    out_type=jax.ShapeDtypeStruct((NSAMP, NCHUNK, CHUNK), jnp.int32),
    compiler_params=pltpu.CompilerParams(needs_layout_passes=False),
    scratch_types=[
        pltpu.VMEM((CHUNK,), jnp.float32),
        pltpu.VMEM((CHUNK,), jnp.int32),
        pltpu.VMEM((256,), jnp.int32),
        pltpu.VMEM((256,), jnp.float32),
        pltpu.VMEM((256,), jnp.float32),
    ],
)
def _otsu_sc(x_hbm, out_hbm, inb, outb, hist, w1f, s1f):
    cid = lax.axis_index("c")
    sid = lax.axis_index("s")
    wid = cid * 16 + sid

    zero16 = jnp.zeros((LANES,), jnp.int32)
    ones16 = jnp.ones((LANES,), jnp.int32)
    iota16 = lax.iota(jnp.int32, LANES)

    for j in range(256 // LANES):
        hist[pl.ds(j * LANES, LANES)] = zero16

    def hist_body(i, carry):
        base = i * LANES * UNROLL
        for u in range(UNROLL):
            xv = inb[pl.ds(base + u * LANES, LANES)]
            idx = (xv * 255.0).astype(jnp.int32)
            plsc.addupdate_scatter(hist, [idx], ones16)
        return carry

    for c in range(NCHUNK):
        pltpu.sync_copy(x_hbm.at[wid, c], inb)
        lax.fori_loop(0, INNER, hist_body, 0)

    w_carry = jnp.int32(0)
    s_carry = jnp.int32(0)
    minx = jnp.int32(1 << 20)
    maxx = jnp.int32(-1)
    for j in range(256 // LANES):
        h = hist[pl.ds(j * LANES, LANES)]
        idxv = iota16 + j * LANES
        w1c = plsc.cumsum(h) + w_carry
        hb = h * idxv
        s1c = plsc.cumsum(hb) + s_carry
        w1f[pl.ds(j * LANES, LANES)] = w1c.astype(jnp.float32)
        s1f[pl.ds(j * LANES, LANES)] = s1c.astype(jnp.float32)
        w_carry = w_carry + jnp.sum(h)
        s_carry = s_carry + jnp.sum(hb)
        nz = h > 0
        minx = jnp.minimum(minx, jnp.min(jnp.where(nz, idxv, 1 << 20)))
        maxx = jnp.maximum(maxx, jnp.max(jnp.where(nz, idxv, -1)))

    n_f = jnp.float32(NPIX)
    s_f = s_carry.astype(jnp.float32)
    minx_f = minx.astype(jnp.float32)
    maxx_f = maxx.astype(jnp.float32)
    best = jnp.float32(-jnp.inf)
    besti = jnp.int32(0)
    for j in range(256 // LANES):
        idxv = iota16 + j * LANES
        tf = idxv.astype(jnp.float32)
        w1v = w1f[pl.ds(j * LANES, LANES)]
        s1v = s1f[pl.ds(j * LANES, LANES)]
        w2v = n_f - w1v
        m1 = s1v / w1v
        m2 = (s_f - s1v) / w2v
        dd = m1 - m2
        var = (w1v * w2v) * (dd * dd)
        valid = (tf >= minx_f) & (tf <= maxx_f - 1.0) & (idxv < 255)
        var = jnp.where(valid, var, -jnp.inf)
        cmax = jnp.max(var)
        cidx = jnp.min(jnp.where(var == cmax, idxv, jnp.int32(512)))
        upd = cmax > best
        besti = jnp.where(upd, cidx, besti)
        best = jnp.where(upd, cmax, best)

    thv = jnp.where(besti == 0, jnp.int32(1), besti)
    thv = jnp.where(thv == 255, jnp.int32(254), thv)
    thv = jnp.where(minx == maxx, jnp.int32(300), thv)

    def out_body(i, carry):
        base = i * LANES * UNROLL
        for u in range(UNROLL):
            xv = inb[pl.ds(base + u * LANES, LANES)]
            idx = (xv * 255.0).astype(jnp.int32)
            outb[pl.ds(base + u * LANES, LANES)] = jnp.where(
                idx > thv, 1, 0
            ).astype(jnp.int32)
        return carry

    for c in range(NCHUNK):
        pltpu.sync_copy(x_hbm.at[wid, c], inb)
        lax.fori_loop(0, INNER, out_body, 0)
        pltpu.sync_copy(outb, out_hbm.at[wid, c])


def kernel(x):
    b, n, h, w = x.shape
    xs = x.reshape(NSAMP, NCHUNK, CHUNK)
    out = _otsu_sc(xs)
    return out.reshape(b, n, h, w).astype(jnp.int64)


# speedup vs baseline: 3114.0733x; 1.1660x over previous
import functools

import jax
import jax.numpy as jnp
from jax import lax
from jax.experimental import pallas as pl
from jax.experimental.pallas import tpu as pltpu
from jax.experimental.pallas import tpu_sc as plsc

H = W = 512
NPIX = H * W
NSAMP = 32
CHUNK = 16384
NCHUNK = NPIX // CHUNK
LANES = 16
UNROLL = 16
INNER = CHUNK // (LANES * UNROLL)

_mesh = plsc.VectorSubcoreMesh(core_axis_name="c", subcore_axis_name="s")


@functools.partial(
    pl.kernel,
    mesh=_mesh,
    out_type=jax.ShapeDtypeStruct((NSAMP, NCHUNK, CHUNK), jnp.int32),
    compiler_params=pltpu.CompilerParams(needs_layout_passes=False),
    scratch_types=[
        pltpu.VMEM((CHUNK,), jnp.float32),
        pltpu.VMEM((CHUNK,), jnp.float32),
        pltpu.VMEM((CHUNK,), jnp.int32),
        pltpu.VMEM((CHUNK,), jnp.int32),
        pltpu.VMEM((256,), jnp.int32),
        pltpu.VMEM((256,), jnp.float32),
        pltpu.VMEM((256,), jnp.float32),
        pltpu.SemaphoreType.DMA,
        pltpu.SemaphoreType.DMA,
    ],
)
def _otsu_sc(x_hbm, out_hbm, ina, inb, outa, outb, hist, w1f, s1f,
             sem_in, sem_out):
    cid = lax.axis_index("c")
    sid = lax.axis_index("s")
    wid = cid * 16 + sid

    zero16 = jnp.zeros((LANES,), jnp.int32)
    ones16 = jnp.ones((LANES,), jnp.int32)
    iota16 = lax.iota(jnp.int32, LANES)
    inbufs = (ina, inb)
    outbufs = (outa, outb)

    for j in range(256 // LANES):
        hist[pl.ds(j * LANES, LANES)] = zero16

    def make_hist_body(buf):
        def hist_body(i, carry):
            base = i * LANES * UNROLL
            for u in range(UNROLL):
                xv = buf[pl.ds(base + u * LANES, LANES)]
                idx = (xv * 255.0).astype(jnp.int32)
                plsc.addupdate_scatter(hist, [idx], ones16)
            return carry
        return hist_body

    copies = [None, None]
    copies[0] = pltpu.async_copy(x_hbm.at[wid, 0], ina, sem_in)
    for c in range(NCHUNK):
        if c + 1 < NCHUNK:
            copies[(c + 1) % 2] = pltpu.async_copy(
                x_hbm.at[wid, c + 1], inbufs[(c + 1) % 2], sem_in)
        copies[c % 2].wait()
        lax.fori_loop(0, INNER, make_hist_body(inbufs[c % 2]), 0)

    copies[0] = pltpu.async_copy(x_hbm.at[wid, 0], ina, sem_in)

    w_carry = jnp.int32(0)
    s_carry = jnp.int32(0)
    minx = jnp.int32(1 << 20)
    maxx = jnp.int32(-1)
    for j in range(256 // LANES):
        h = hist[pl.ds(j * LANES, LANES)]
        idxv = iota16 + j * LANES
        w1c = plsc.cumsum(h) + w_carry
        hb = h * idxv
        s1c = plsc.cumsum(hb) + s_carry
        w1f[pl.ds(j * LANES, LANES)] = w1c.astype(jnp.float32)
        s1f[pl.ds(j * LANES, LANES)] = s1c.astype(jnp.float32)
        w_carry = w_carry + jnp.sum(h)
        s_carry = s_carry + jnp.sum(hb)
        nz = h > 0
        minx = jnp.minimum(minx, jnp.min(jnp.where(nz, idxv, 1 << 20)))
        maxx = jnp.maximum(maxx, jnp.max(jnp.where(nz, idxv, -1)))

    n_f = jnp.float32(NPIX)
    s_f = s_carry.astype(jnp.float32)
    minx_f = minx.astype(jnp.float32)
    maxx_f = maxx.astype(jnp.float32)
    best = jnp.float32(-jnp.inf)
    besti = jnp.int32(0)
    for j in range(256 // LANES):
        idxv = iota16 + j * LANES
        tf = idxv.astype(jnp.float32)
        w1v = w1f[pl.ds(j * LANES, LANES)]
        s1v = s1f[pl.ds(j * LANES, LANES)]
        w2v = n_f - w1v
        m1 = s1v / w1v
        m2 = (s_f - s1v) / w2v
        dd = m1 - m2
        var = (w1v * w2v) * (dd * dd)
        valid = (tf >= minx_f) & (tf <= maxx_f - 1.0) & (idxv < 255)
        var = jnp.where(valid, var, -jnp.inf)
        cmax = jnp.max(var)
        cidx = jnp.min(jnp.where(var == cmax, idxv, jnp.int32(512)))
        upd = cmax > best
        besti = jnp.where(upd, cidx, besti)
        best = jnp.where(upd, cmax, best)

    thv = jnp.where(besti == 0, jnp.int32(1), besti)
    thv = jnp.where(thv == 255, jnp.int32(254), thv)
    thv = jnp.where(minx == maxx, jnp.int32(300), thv)
    cut = (thv + 1).astype(jnp.float32)

    def make_out_body(bufi, bufo):
        def out_body(i, carry):
            base = i * LANES * UNROLL
            for u in range(UNROLL):
                xv = bufi[pl.ds(base + u * LANES, LANES)]
                bufo[pl.ds(base + u * LANES, LANES)] = jnp.where(
                    xv * 255.0 >= cut, jnp.int32(1), jnp.int32(0))
            return carry
        return out_body

    out_copies = [None, None]
    for c in range(NCHUNK):
        if c + 1 < NCHUNK:
            copies[(c + 1) % 2] = pltpu.async_copy(
                x_hbm.at[wid, c + 1], inbufs[(c + 1) % 2], sem_in)
        copies[c % 2].wait()
        if c >= 2:
            out_copies[c % 2].wait()
        lax.fori_loop(0, INNER,
                      make_out_body(inbufs[c % 2], outbufs[c % 2]), 0)
        out_copies[c % 2] = pltpu.async_copy(
            outbufs[c % 2], out_hbm.at[wid, c], sem_out)
    out_copies[0].wait()
    out_copies[1].wait()


def kernel(x):
    b, n, h, w = x.shape
    xs = x.reshape(NSAMP, NCHUNK, CHUNK)
    out = _otsu_sc(xs)
    return out.reshape(b, n, h, w).astype(jnp.int64)


# speedup vs baseline: 5841.3676x; 1.8758x over previous
import functools

import jax
import jax.numpy as jnp
from jax import lax
from jax.experimental import pallas as pl
from jax.experimental.pallas import tpu as pltpu
from jax.experimental.pallas import tpu_sc as plsc

H = W = 512
NPIX = H * W
NSAMP = 32
CHUNK = 16384
NCHUNK = NPIX // CHUNK
LANES = 16
UNROLL = 16
INNER = CHUNK // (LANES * UNROLL)

_mesh = plsc.VectorSubcoreMesh(core_axis_name="c", subcore_axis_name="s")


@functools.partial(
    pl.kernel,
    mesh=_mesh,
    out_type=jax.ShapeDtypeStruct((NSAMP, NCHUNK, CHUNK), jnp.int32),
    compiler_params=pltpu.CompilerParams(needs_layout_passes=False),
    scratch_types=[
        pltpu.VMEM((CHUNK,), jnp.float32),
        pltpu.VMEM((CHUNK,), jnp.float32),
        pltpu.VMEM((CHUNK,), jnp.int32),
        pltpu.VMEM((CHUNK,), jnp.int32),
        pltpu.VMEM((256,), jnp.int32),
        pltpu.VMEM((256,), jnp.float32),
        pltpu.VMEM((256,), jnp.float32),
        pltpu.SemaphoreType.DMA,
        pltpu.SemaphoreType.DMA,
    ],
)
def _otsu_sc(x_hbm, out_hbm, ina, inb, outa, outb, hist, w1f, s1f,
             sem_in, sem_out):
    cid = lax.axis_index("c")
    sid = lax.axis_index("s")
    wid = cid * 16 + sid

    zero16 = jnp.zeros((LANES,), jnp.int32)
    ones16 = jnp.ones((LANES,), jnp.int32)
    iota16 = lax.iota(jnp.int32, LANES)
    inbufs = (ina, inb)
    outbufs = (outa, outb)

    for j in range(256 // LANES):
        hist[pl.ds(j * LANES, LANES)] = zero16

    def make_hist_body(buf):
        def hist_body(i, carry):
            base = i * LANES * UNROLL
            xs = [buf[pl.ds(base + u * LANES, LANES)]
                  for u in range(UNROLL)]
            idxs = [(xv * 255.0).astype(jnp.int32) for xv in xs]
            for idx in idxs:
                plsc.addupdate_scatter(hist, [idx], ones16)
            return carry
        return hist_body

    copies = [None, None]
    copies[0] = pltpu.async_copy(x_hbm.at[wid, 0], ina, sem_in)
    for c in range(NCHUNK):
        if c + 1 < NCHUNK:
            copies[(c + 1) % 2] = pltpu.async_copy(
                x_hbm.at[wid, c + 1], inbufs[(c + 1) % 2], sem_in)
        copies[c % 2].wait()
        lax.fori_loop(0, INNER, make_hist_body(inbufs[c % 2]), 0)

    copies[0] = pltpu.async_copy(x_hbm.at[wid, 0], ina, sem_in)

    w_carry = jnp.int32(0)
    s_carry = jnp.int32(0)
    minx = jnp.int32(1 << 20)
    maxx = jnp.int32(-1)
    for j in range(256 // LANES):
        h = hist[pl.ds(j * LANES, LANES)]
        idxv = iota16 + j * LANES
        w1c = plsc.cumsum(h) + w_carry
        hb = h * idxv
        s1c = plsc.cumsum(hb) + s_carry
        w1f[pl.ds(j * LANES, LANES)] = w1c.astype(jnp.float32)
        s1f[pl.ds(j * LANES, LANES)] = s1c.astype(jnp.float32)
        w_carry = w_carry + jnp.sum(h)
        s_carry = s_carry + jnp.sum(hb)
        nz = h > 0
        minx = jnp.minimum(minx, jnp.min(jnp.where(nz, idxv, 1 << 20)))
        maxx = jnp.maximum(maxx, jnp.max(jnp.where(nz, idxv, -1)))

    n_f = jnp.float32(NPIX)
    s_f = s_carry.astype(jnp.float32)
    minx_f = minx.astype(jnp.float32)
    maxx_f = maxx.astype(jnp.float32)
    best = jnp.float32(-jnp.inf)
    besti = jnp.int32(0)
    for j in range(256 // LANES):
        idxv = iota16 + j * LANES
        tf = idxv.astype(jnp.float32)
        w1v = w1f[pl.ds(j * LANES, LANES)]
        s1v = s1f[pl.ds(j * LANES, LANES)]
        w2v = n_f - w1v
        m1 = s1v / w1v
        m2 = (s_f - s1v) / w2v
        dd = m1 - m2
        var = (w1v * w2v) * (dd * dd)
        valid = (tf >= minx_f) & (tf <= maxx_f - 1.0) & (idxv < 255)
        var = jnp.where(valid, var, -jnp.inf)
        cmax = jnp.max(var)
        cidx = jnp.min(jnp.where(var == cmax, idxv, jnp.int32(512)))
        upd = cmax > best
        besti = jnp.where(upd, cidx, besti)
        best = jnp.where(upd, cmax, best)

    thv = jnp.where(besti == 0, jnp.int32(1), besti)
    thv = jnp.where(thv == 255, jnp.int32(254), thv)
    thv = jnp.where(minx == maxx, jnp.int32(300), thv)
    cut = (thv + 1).astype(jnp.float32)

    def make_out_body(bufi, bufo):
        def out_body(i, carry):
            base = i * LANES * UNROLL
            xs = [bufi[pl.ds(base + u * LANES, LANES)]
                  for u in range(UNROLL)]
            rois = [jnp.where(xv * 255.0 >= cut, jnp.int32(1), jnp.int32(0))
                    for xv in xs]
            for u in range(UNROLL):
                bufo[pl.ds(base + u * LANES, LANES)] = rois[u]
            return carry
        return out_body

    out_copies = [None, None]
    for c in range(NCHUNK):
        if c + 1 < NCHUNK:
            copies[(c + 1) % 2] = pltpu.async_copy(
                x_hbm.at[wid, c + 1], inbufs[(c + 1) % 2], sem_in)
        copies[c % 2].wait()
        if c >= 2:
            out_copies[c % 2].wait()
        lax.fori_loop(0, INNER,
                      make_out_body(inbufs[c % 2], outbufs[c % 2]), 0)
        out_copies[c % 2] = pltpu.async_copy(
            outbufs[c % 2], out_hbm.at[wid, c], sem_out)
    out_copies[0].wait()
    out_copies[1].wait()


def kernel(x):
    b, n, h, w = x.shape
    xs = x.reshape(NSAMP, NCHUNK, CHUNK)
    out = _otsu_sc(xs)
    return out.reshape(b, n, h, w).astype(jnp.int64)
